# baseline (speedup 1.0000x reference)
import jax
import jax.numpy as jnp
from jax import lax
from jax.experimental import pallas as pl
from jax.experimental.pallas import tpu as pltpu
from jax.experimental.pallas import tpu_sc as plsc

U = 50000
I = 50000
D = 32
N_LAYERS = 3

NCORE = 2
NSUB = 16
LANES = 16

CH = 384
SB = 128
NSB = CH // SB
R = 50000
RT = R + 8
NPAD = 2 * RT

SEG = 3128
SEG_LAST = RT - 15 * SEG


def _cdiv(a, b):
    return (a + b - 1) // b


def _layer_kernel_body(xp_hbm, src_hbm, dst_hbm, w_hbm, zc_hbm, out_hbm,
                       src_v0, src_v1, draw_v0, draw_v1, w_v0, w_v1,
                       csrc_v0, csrc_v1, sidx_v0, sidx_v1,
                       rows_v0, rows_v1, nb_smem,
                       acc_shared, sem_e, sem_g, sem_s0, sem_s1):
    c = lax.axis_index("c")
    s = lax.axis_index("s")
    coff = c * R
    k_chunks = src_hbm.shape[0] // (NSUB * CH)

    src_v = (src_v0, src_v1)
    draw_v = (draw_v0, draw_v1)
    w_v = (w_v0, w_v1)
    csrc_v = (csrc_v0, csrc_v1)
    sidx_v = (sidx_v0, sidx_v1)
    rows_v = (rows_v0, rows_v1)
    sem_s = (sem_s0, sem_s1)

    seg_start = s * SEG

    @pl.when(s < 15)
    def _():
        pltpu.sync_copy(zc_hbm.at[pl.ds(0, SEG)],
                        acc_shared.at[pl.ds(seg_start, SEG)])

    @pl.when(s == 15)
    def _():
        pltpu.sync_copy(zc_hbm.at[pl.ds(0, SEG_LAST)],
                        acc_shared.at[pl.ds(seg_start, SEG_LAST)])

    plsc.subcore_barrier()

    def fire_edges(k, b):
        ebase = (s * k_chunks + k) * CH
        pltpu.async_copy(src_hbm.at[pl.ds(ebase, CH)], src_v[b], sem_e)
        pltpu.async_copy(dst_hbm.at[pl.ds(ebase, CH)], draw_v[b], sem_e)
        pltpu.async_copy(w_hbm.at[pl.ds(ebase, CH)], w_v[b], sem_e)

    def wait_edges(b):
        pltpu.make_async_copy(src_hbm.at[pl.ds(0, CH)], src_v[b], sem_e).wait()
        pltpu.make_async_copy(dst_hbm.at[pl.ds(0, CH)], draw_v[b], sem_e).wait()
        pltpu.make_async_copy(w_hbm.at[pl.ds(0, CH)], w_v[b], sem_e).wait()

    def fire_gathers(b, nb):
        for j in range(NSB):
            @pl.when(j < nb)
            def _():
                pltpu.async_copy(xp_hbm.at[csrc_v[b].at[j]],
                                 rows_v[b].at[pl.ds(j * SB, SB)], sem_g)

    def wait_gathers(b, nb):
        for j in range(NSB):
            @pl.when(j < nb)
            def _():
                pltpu.make_async_copy(
                    xp_hbm.at[csrc_v[b].at[j]],
                    rows_v[b].at[pl.ds(j * SB, SB)], sem_g).wait()

    def fire_scatters(b, nb):
        for j in range(NSB):
            @pl.when(j < nb)
            def _():
                pltpu.async_copy(rows_v[b].at[pl.ds(j * SB, SB)],
                                 acc_shared.at[sidx_v[b].at[j]], sem_s[b],
                                 add=True)

    def wait_scatters(b, nb):
        for j in range(NSB):
            @pl.when(j < nb)
            def _():
                pltpu.make_async_copy(rows_v[b].at[pl.ds(j * SB, SB)],
                                      acc_shared.at[sidx_v[b].at[j]],
                                      sem_s[b]).wait()

    def compact(b):
        cnt = jnp.int32(0)
        for g in range(CH // LANES):
            d = draw_v[b][pl.ds(g * LANES, LANES)]
            s16 = src_v[b][pl.ds(g * LANES, LANES)]
            w16 = w_v[b][pl.ds(g * LANES, LANES)]
            dl = d - coff
            ok = (dl >= 0) & (dl < R)
            oki = ok.astype(jnp.int32)
            cs = plsc.cumsum(oki)
            pos = cnt + cs - oki
            prow = pos >> 7
            pcol = pos & 127
            plsc.store_scatter(sidx_v[b], [prow, pcol], dl, mask=ok)
            plsc.store_scatter(csrc_v[b], [prow, pcol], s16, mask=ok)
            plsc.store_scatter(w_v[b], [pos], w16, mask=ok)
            cnt = cnt + cs[15]
        nb = (cnt + (SB - 1)) >> 7
        cnt_pad = nb << 7
        lane = lax.iota(jnp.int32, LANES)
        zi = jnp.zeros((LANES,), jnp.int32)
        zf = jnp.zeros((LANES,), jnp.float32)
        trash = jnp.full((LANES,), R, jnp.int32)
        for g in range(SB // LANES):
            pos = cnt + g * LANES + lane
            msk = pos < cnt_pad
            prow = pos >> 7
            pcol = pos & 127
            plsc.store_scatter(sidx_v[b], [prow, pcol], trash, mask=msk)
            plsc.store_scatter(csrc_v[b], [prow, pcol], zi, mask=msk)
            plsc.store_scatter(w_v[b], [pos], zf, mask=msk)
        nb_smem[b] = nb
        return nb

    def scale(b, nb):
        for j in range(NSB):
            @pl.when(j < nb)
            def _():
                for g in range(SB // LANES):
                    w16 = w_v[b][pl.ds(j * SB + g * LANES, LANES)]
                    for i in range(LANES):
                        wi = w16[i]
                        e = j * SB + g * LANES + i
                        for cc in range(D // LANES):
                            sl = (e, pl.ds(cc * LANES, LANES))
                            rows_v[b][sl] = rows_v[b][sl] * wi

    fire_edges(0, 0)

    def iter_k(k, b):
        pb = 1 - b
        wait_edges(b)

        @pl.when(k >= 1)
        def _():
            nb_p = nb_smem[pb]
            wait_gathers(pb, nb_p)
            scale(pb, nb_p)
            fire_scatters(pb, nb_p)

        @pl.when(k <= k_chunks - 2)
        def _():
            fire_edges(k + 1, pb)

        @pl.when(k >= 2)
        def _():
            wait_scatters(b, nb_smem[b])

        nb = compact(b)
        fire_gathers(b, nb)

    @pl.loop(0, k_chunks, step=2)
    def _(k):
        iter_k(k, 0)
        iter_k(k + 1, 1)

    nb1 = nb_smem[1]
    wait_gathers(1, nb1)
    scale(1, nb1)
    fire_scatters(1, nb1)
    wait_scatters(0, nb_smem[0])
    wait_scatters(1, nb1)

    plsc.subcore_barrier()

    out_base = c * RT + seg_start

    @pl.when(s < 15)
    def _():
        pltpu.sync_copy(acc_shared.at[pl.ds(seg_start, SEG)],
                        out_hbm.at[pl.ds(out_base, SEG)])

    @pl.when(s == 15)
    def _():
        pltpu.sync_copy(acc_shared.at[pl.ds(seg_start, SEG_LAST)],
                        out_hbm.at[pl.ds(out_base, SEG_LAST)])


_MESH = plsc.VectorSubcoreMesh(core_axis_name="c", subcore_axis_name="s",
                               num_cores=NCORE, num_subcores=NSUB)

_layer = pl.kernel(
    _layer_kernel_body,
    out_type=jax.ShapeDtypeStruct((NPAD, D), jnp.float32),
    mesh=_MESH,
    scratch_types=[
        pltpu.VMEM((CH,), jnp.int32),
        pltpu.VMEM((CH,), jnp.int32),
        pltpu.VMEM((CH,), jnp.int32),
        pltpu.VMEM((CH,), jnp.int32),
        pltpu.VMEM((CH,), jnp.float32),
        pltpu.VMEM((CH,), jnp.float32),
        pltpu.VMEM((NSB, SB), jnp.int32),
        pltpu.VMEM((NSB, SB), jnp.int32),
        pltpu.VMEM((NSB, SB), jnp.int32),
        pltpu.VMEM((NSB, SB), jnp.int32),
        pltpu.VMEM((CH, D), jnp.float32),
        pltpu.VMEM((CH, D), jnp.float32),
        pltpu.SMEM((2,), jnp.int32),
        pltpu.VMEM_SHARED((RT, D), jnp.float32),
        pltpu.SemaphoreType.DMA,
        pltpu.SemaphoreType.DMA,
        pltpu.SemaphoreType.DMA,
        pltpu.SemaphoreType.DMA,
    ],
    compiler_params=pltpu.CompilerParams(use_tc_tiling_on_sc=False,
                                         needs_layout_passes=False),
)


def kernel(u_emb, i_emb, edge_index, edge_weight):
    E = edge_index.shape[1]
    chunks = _cdiv(E, NSUB * CH)
    chunks += chunks % 2
    e_pad = chunks * NSUB * CH

    src = edge_index[0].astype(jnp.int32)
    dst = edge_index[1].astype(jnp.int32)
    src = src + 8 * (src >= U).astype(jnp.int32)
    pad = e_pad - E
    src = jnp.concatenate([src, jnp.zeros((pad,), jnp.int32)])
    dst = jnp.concatenate([dst, jnp.full((pad,), 2 * R, jnp.int32)])
    w = jnp.concatenate([edge_weight, jnp.zeros((pad,), jnp.float32)])

    zc = jnp.zeros((SEG, D), jnp.float32)
    zpad = jnp.zeros((8, D), jnp.float32)
    xp = jnp.concatenate([u_emb, zpad, i_emb, zpad], axis=0)

    acc = xp
    x = xp
    for _ in range(N_LAYERS):
        x = _layer(x, src, dst, w, zc)
        acc = acc + x
    final = acc * (1.0 / (N_LAYERS + 1))
    return (final[:U], final[RT:RT + I])


# speedup vs baseline: 4.3362x; 4.3362x over previous
import jax
import jax.numpy as jnp
from jax import lax
from jax.experimental import pallas as pl
from jax.experimental.pallas import tpu as pltpu
from jax.experimental.pallas import tpu_sc as plsc

U = 50000
I = 50000
D = 32
N_LAYERS = 3

NCORE = 2
NSUB = 16
LANES = 16

CH = 384
SB = 128
NSB = CH // SB
R = 50000
RT = R + 8
NPAD = 2 * RT

SEG = 3128
SEG_LAST = RT - 15 * SEG


def _cdiv(a, b):
    return (a + b - 1) // b


def _layer_kernel_body(xp_hbm, src_hbm, dst_hbm, w_hbm, out_hbm,
                       src_v0, src_v1, draw_v0, draw_v1, w_v0, w_v1,
                       sidx_v0, sidx_v1, rows_v0, rows_v1, zero_v,
                       acc_shared, sem_e, sem_g, sem_s0, sem_s1):
    c = lax.axis_index("c")
    s = lax.axis_index("s")
    coff = c * R
    k_chunks = src_hbm.shape[0] // (NSUB * NSB)

    src_v = (src_v0, src_v1)
    draw_v = (draw_v0, draw_v1)
    w_v = (w_v0, w_v1)
    sidx_v = (sidx_v0, sidx_v1)
    rows_v = (rows_v0, rows_v1)
    sem_s = (sem_s0, sem_s1)

    for r in range(8):
        for cc in range(D // LANES):
            zero_v[pl.ds(r, 1), pl.ds(cc * LANES, LANES)] = jnp.zeros(
                (1, LANES), jnp.float32)
    seg_start = s * SEG

    @pl.when(s < 15)
    def _():
        @pl.loop(0, SEG // 8)
        def _(z):
            pltpu.sync_copy(zero_v, acc_shared.at[pl.ds(seg_start + z * 8, 8)])

    @pl.when(s == 15)
    def _():
        @pl.loop(0, SEG_LAST // 8)
        def _(z):
            pltpu.sync_copy(zero_v, acc_shared.at[pl.ds(seg_start + z * 8, 8)])

    plsc.subcore_barrier()

    def fire_edges(k, b):
        rbase = (s * k_chunks + k) * NSB
        ebase = rbase * SB
        pltpu.async_copy(src_hbm.at[pl.ds(rbase, NSB)], src_v[b], sem_e)
        pltpu.async_copy(dst_hbm.at[pl.ds(rbase, NSB)], draw_v[b], sem_e)
        pltpu.async_copy(w_hbm.at[pl.ds(ebase, CH)], w_v[b], sem_e)

    def wait_edges(b):
        pltpu.make_async_copy(src_hbm.at[pl.ds(0, NSB)], src_v[b], sem_e).wait()
        pltpu.make_async_copy(dst_hbm.at[pl.ds(0, NSB)], draw_v[b], sem_e).wait()
        pltpu.make_async_copy(w_hbm.at[pl.ds(0, CH)], w_v[b], sem_e).wait()

    def fire_gathers(b):
        for j in range(NSB):
            pltpu.async_copy(xp_hbm.at[src_v[b].at[j]],
                             rows_v[b].at[pl.ds(j * SB, SB)], sem_g)

    def wait_gathers(b):
        for j in range(NSB):
            pltpu.make_async_copy(xp_hbm.at[src_v[b].at[j]],
                                  rows_v[b].at[pl.ds(j * SB, SB)], sem_g).wait()

    def fire_scatters(b):
        for j in range(NSB):
            pltpu.async_copy(rows_v[b].at[pl.ds(j * SB, SB)],
                             acc_shared.at[sidx_v[b].at[j]], sem_s[b], add=True)

    def wait_scatters(b):
        for j in range(NSB):
            pltpu.make_async_copy(rows_v[b].at[pl.ds(j * SB, SB)],
                                  acc_shared.at[sidx_v[b].at[j]],
                                  sem_s[b]).wait()

    def transform_dst(b):
        @pl.loop(0, NSB)
        def _(r):
            for cc in range(SB // LANES):
                sl = (pl.ds(r, 1), pl.ds(cc * LANES, LANES))
                d = draw_v[b][sl]
                dl = d - coff
                ok = (dl >= 0) & (dl < R)
                sidx_v[b][sl] = jnp.where(ok, dl, R)

    def scale(b):
        @pl.loop(0, CH, step=LANES)
        def _(e0):
            w16 = w_v[b][pl.ds(e0, LANES)]
            for i in range(LANES):
                wi = w16[i]
                for cc in range(D // LANES):
                    sl = (pl.ds(e0 + i, 1), pl.ds(cc * LANES, LANES))
                    rows_v[b][sl] = rows_v[b][sl] * wi

    fire_edges(0, 0)

    def iter_k(k, b):
        pb = 1 - b
        wait_edges(b)

        @pl.when(k >= 1)
        def _():
            wait_gathers(pb)
            scale(pb)
            fire_scatters(pb)

        @pl.when(k <= k_chunks - 2)
        def _():
            fire_edges(k + 1, pb)

        @pl.when(k >= 2)
        def _():
            wait_scatters(b)

        transform_dst(b)
        fire_gathers(b)

    @pl.loop(0, k_chunks, step=2)
    def _(k):
        iter_k(k, 0)
        iter_k(k + 1, 1)

    wait_gathers(1)
    scale(1)
    fire_scatters(1)
    wait_scatters(0)
    wait_scatters(1)

    plsc.subcore_barrier()

    out_base = c * RT + seg_start

    @pl.when(s < 15)
    def _():
        pltpu.sync_copy(acc_shared.at[pl.ds(seg_start, SEG)],
                        out_hbm.at[pl.ds(out_base, SEG)])

    @pl.when(s == 15)
    def _():
        pltpu.sync_copy(acc_shared.at[pl.ds(seg_start, SEG_LAST)],
                        out_hbm.at[pl.ds(out_base, SEG_LAST)])


_MESH = plsc.VectorSubcoreMesh(core_axis_name="c", subcore_axis_name="s",
                               num_cores=NCORE, num_subcores=NSUB)

_layer = pl.kernel(
    _layer_kernel_body,
    out_type=jax.ShapeDtypeStruct((NPAD, D), jnp.float32),
    mesh=_MESH,
    scratch_types=[
        pltpu.VMEM((NSB, SB), jnp.int32),
        pltpu.VMEM((NSB, SB), jnp.int32),
        pltpu.VMEM((NSB, SB), jnp.int32),
        pltpu.VMEM((NSB, SB), jnp.int32),
        pltpu.VMEM((CH,), jnp.float32),
        pltpu.VMEM((CH,), jnp.float32),
        pltpu.VMEM((NSB, SB), jnp.int32),
        pltpu.VMEM((NSB, SB), jnp.int32),
        pltpu.VMEM((CH, D), jnp.float32),
        pltpu.VMEM((CH, D), jnp.float32),
        pltpu.VMEM((8, D), jnp.float32),
        pltpu.VMEM_SHARED((RT, D), jnp.float32),
        pltpu.SemaphoreType.DMA,
        pltpu.SemaphoreType.DMA,
        pltpu.SemaphoreType.DMA,
        pltpu.SemaphoreType.DMA,
    ],
    compiler_params=pltpu.CompilerParams(use_tc_tiling_on_sc=False),
)


def kernel(u_emb, i_emb, edge_index, edge_weight):
    E = edge_index.shape[1]
    chunks = _cdiv(E, NSUB * CH)
    chunks += chunks % 2
    e_pad = chunks * NSUB * CH

    src = edge_index[0].astype(jnp.int32)
    dst = edge_index[1].astype(jnp.int32)
    src = src + 8 * (src >= U).astype(jnp.int32)
    pad = e_pad - E
    src = jnp.concatenate([src, jnp.zeros((pad,), jnp.int32)])
    dst = jnp.concatenate([dst, jnp.zeros((pad,), jnp.int32)])
    w = jnp.concatenate([edge_weight, jnp.zeros((pad,), jnp.float32)])
    src2d = src.reshape(e_pad // SB, SB)
    dst2d = dst.reshape(e_pad // SB, SB)

    zpad = jnp.zeros((8, D), jnp.float32)
    xp = jnp.concatenate([u_emb, zpad, i_emb, zpad], axis=0)

    acc = xp
    x = xp
    for _ in range(N_LAYERS):
        x = _layer(x, src2d, dst2d, w)
        acc = acc + x
    final = acc * (1.0 / (N_LAYERS + 1))
    return (final[:U], final[RT:RT + I])
